# initial kernel scaffold (unmeasured)
import jax
import jax.numpy as jnp
from jax import lax
from jax.experimental import pallas as pl
from jax.experimental.pallas import tpu as pltpu

B, S, D = 2, 512, 2048
DC = 128
H, DH, DR = 16, 128, 32
SCALE = (DH + DR) ** -0.5
F32 = jnp.float32


def kernel(x, Wdkv, Wuk, Wuv, Wq, Wqr, Wkr, Wo):
    def body(x_ref, wdkv_ref, wuk_ref, wuv_ref, wq_ref, wqr_ref, wkr_ref,
             wo_ref, out_ref,
             c_ref, c_rx_ref, wuk_rx_ref, wuv_rx_ref,
             k_ref, v_ref, q_ref, qr_ref, kr_ref, o_ref,
             send_sems, recv_sems):
        my_x = lax.axis_index("x")
        my_y = lax.axis_index("y")
        my_z = lax.axis_index("z")
        peer = (1 - my_x, my_y, my_z)

        barrier_sem = pltpu.get_barrier_semaphore()
        pl.semaphore_signal(barrier_sem, inc=1, device_id=peer,
                            device_id_type=pl.DeviceIdType.MESH)
        pl.semaphore_wait(barrier_sem, 1)

        x2d = x_ref[...].reshape(B * S, D)

        c_ref[...] = jnp.dot(x2d, wdkv_ref[...], preferred_element_type=F32)

        rdma_c = pltpu.make_async_remote_copy(
            src_ref=c_ref, dst_ref=c_rx_ref,
            send_sem=send_sems.at[0], recv_sem=recv_sems.at[0],
            device_id=peer, device_id_type=pl.DeviceIdType.MESH)
        rdma_wuk = pltpu.make_async_remote_copy(
            src_ref=wuk_ref, dst_ref=wuk_rx_ref,
            send_sem=send_sems.at[1], recv_sem=recv_sems.at[1],
            device_id=peer, device_id_type=pl.DeviceIdType.MESH)
        rdma_wuv = pltpu.make_async_remote_copy(
            src_ref=wuv_ref, dst_ref=wuv_rx_ref,
            send_sem=send_sems.at[2], recv_sem=recv_sems.at[2],
            device_id=peer, device_id_type=pl.DeviceIdType.MESH)
        rdma_c.start()
        rdma_wuk.start()
        rdma_wuv.start()

        k_ref[...] = jnp.dot(c_ref[...], wuk_ref[...], preferred_element_type=F32)
        v_ref[...] = jnp.dot(c_ref[...], wuv_ref[...], preferred_element_type=F32)
        q_ref[...] = jnp.dot(x2d, wq_ref[...], preferred_element_type=F32)
        qr_ref[...] = jnp.dot(x2d, wqr_ref[...], preferred_element_type=F32)
        kr_ref[...] = jnp.dot(x2d, wkr_ref[...], preferred_element_type=F32)

        rdma_c.wait()
        rdma_wuk.wait()
        rdma_wuv.wait()

        k_ref[...] += jnp.dot(c_rx_ref[...], wuk_rx_ref[...],
                              preferred_element_type=F32)
        v_ref[...] += jnp.dot(c_rx_ref[...], wuv_rx_ref[...],
                              preferred_element_type=F32)

        for h in range(H):
            hd = slice(h * DH, (h + 1) * DH)
            hr = slice(h * DR, (h + 1) * DR)
            for b in range(B):
                rows = slice(b * S, (b + 1) * S)
                q = q_ref[rows, hd]
                k = k_ref[rows, hd]
                qr = qr_ref[rows, hr]
                kr = kr_ref[rows, :]
                s = (lax.dot_general(q, k, (((1,), (1,)), ((), ())),
                                     preferred_element_type=F32)
                     + lax.dot_general(qr, kr, (((1,), (1,)), ((), ())),
                                       preferred_element_type=F32)) * SCALE
                m = jnp.max(s, axis=-1, keepdims=True)
                p = jnp.exp(s - m)
                p = p / jnp.sum(p, axis=-1, keepdims=True)
                o_ref[rows, hd] = jnp.dot(p, v_ref[rows, hd],
                                          preferred_element_type=F32)

        out2d = jnp.dot(o_ref[...], wo_ref[...], preferred_element_type=F32)
        out_ref[...] = out2d.reshape(B, S, D)

    return pl.pallas_call(
        body,
        out_shape=jax.ShapeDtypeStruct((B, S, D), F32),
        in_specs=[pl.BlockSpec(memory_space=pltpu.VMEM)] * 8,
        out_specs=pl.BlockSpec(memory_space=pltpu.VMEM),
        scratch_shapes=[
            pltpu.VMEM((B * S, DC), F32),
            pltpu.VMEM((B * S, DC), F32),
            pltpu.VMEM((DC, D), F32),
            pltpu.VMEM((DC, D), F32),
            pltpu.VMEM((B * S, D), F32),
            pltpu.VMEM((B * S, D), F32),
            pltpu.VMEM((B * S, D), F32),
            pltpu.VMEM((B * S, H * DR), F32),
            pltpu.VMEM((B * S, DR), F32),
            pltpu.VMEM((B * S, D), F32),
            pltpu.SemaphoreType.DMA((3,)),
            pltpu.SemaphoreType.DMA((3,)),
        ],
        compiler_params=pltpu.CompilerParams(collective_id=0),
    )(x, Wdkv, Wuk, Wuv, Wq, Wqr, Wkr, Wo)


# baseline (device time: 154171 ns/iter reference)
import jax
import jax.numpy as jnp
from jax import lax
from jax.experimental import pallas as pl
from jax.experimental.pallas import tpu as pltpu

B, S, D = 2, 512, 2048
DC = 128
H, DH, DR = 16, 128, 32
SCALE = (DH + DR) ** -0.5
F32 = jnp.float32


def _kv_collective(x, Wdkv, Wuk, Wuv, Wkr):

    def body(x_ref, wdkv_ref, wuk_ref, wuv_ref, wkr_ref,
             k_ref, v_ref, kr_ref,
             c_ref, c_rx_ref, wuk_rx_ref, wuv_rx_ref,
             send_sems, recv_sems):
        my_x = lax.axis_index("x")
        my_y = lax.axis_index("y")
        my_z = lax.axis_index("z")
        peer = (1 - my_x, my_y, my_z)

        barrier_sem = pltpu.get_barrier_semaphore()
        pl.semaphore_signal(barrier_sem, inc=1, device_id=peer,
                            device_id_type=pl.DeviceIdType.MESH)
        pl.semaphore_wait(barrier_sem, 1)

        for b in range(B):
            rows = slice(b * S, (b + 1) * S)
            c_ref[rows, :] = jnp.dot(x_ref[b], wdkv_ref[...],
                                     preferred_element_type=F32)

        rdma_c = pltpu.make_async_remote_copy(
            src_ref=c_ref, dst_ref=c_rx_ref,
            send_sem=send_sems.at[0], recv_sem=recv_sems.at[0],
            device_id=peer, device_id_type=pl.DeviceIdType.MESH)
        rdma_wuk = pltpu.make_async_remote_copy(
            src_ref=wuk_ref, dst_ref=wuk_rx_ref,
            send_sem=send_sems.at[1], recv_sem=recv_sems.at[1],
            device_id=peer, device_id_type=pl.DeviceIdType.MESH)
        rdma_wuv = pltpu.make_async_remote_copy(
            src_ref=wuv_ref, dst_ref=wuv_rx_ref,
            send_sem=send_sems.at[2], recv_sem=recv_sems.at[2],
            device_id=peer, device_id_type=pl.DeviceIdType.MESH)
        rdma_c.start()
        rdma_wuk.start()
        rdma_wuv.start()

        NCH = 4
        CW = D // NCH
        for j in range(NCH):
            cols = slice(j * CW, (j + 1) * CW)
            k_ref[:, cols] = jnp.dot(c_ref[...], wuk_ref[:, cols],
                                     preferred_element_type=F32)
            v_ref[:, cols] = jnp.dot(c_ref[...], wuv_ref[:, cols],
                                     preferred_element_type=F32)
        for b in range(B):
            rows = slice(b * S, (b + 1) * S)
            kr_ref[rows, :] = jnp.dot(x_ref[b], wkr_ref[...],
                                      preferred_element_type=F32)

        rdma_c.wait()
        rdma_wuk.wait()
        rdma_wuv.wait()

        for j in range(NCH):
            cols = slice(j * CW, (j + 1) * CW)
            k_ref[:, cols] += jnp.dot(c_rx_ref[...], wuk_rx_ref[:, cols],
                                      preferred_element_type=F32)
            v_ref[:, cols] += jnp.dot(c_rx_ref[...], wuv_rx_ref[:, cols],
                                      preferred_element_type=F32)

    return pl.pallas_call(
        body,
        out_shape=(
            jax.ShapeDtypeStruct((B * S, D), F32),
            jax.ShapeDtypeStruct((B * S, D), F32),
            jax.ShapeDtypeStruct((B * S, DR), F32),
        ),
        in_specs=[pl.BlockSpec(memory_space=pltpu.VMEM)] * 5,
        out_specs=(pl.BlockSpec(memory_space=pltpu.VMEM),) * 3,
        scratch_shapes=[
            pltpu.VMEM((B * S, DC), F32),
            pltpu.VMEM((B * S, DC), F32),
            pltpu.VMEM((DC, D), F32),
            pltpu.VMEM((DC, D), F32),
            pltpu.SemaphoreType.DMA((3,)),
            pltpu.SemaphoreType.DMA((3,)),
        ],
        compiler_params=pltpu.CompilerParams(collective_id=0),
    )(x, Wdkv, Wuk, Wuv, Wkr)


def _attention(x, Wq, Wqr_t, K, V, Kr):

    def body(x_ref, wq_ref, wqr_ref, k_ref, v_ref, kr_ref, o_ref):
        xb = x_ref[0]
        q = jnp.dot(xb, wq_ref[...], preferred_element_type=F32)
        qr = jnp.dot(xb, wqr_ref[0], preferred_element_type=F32)
        s = (lax.dot_general(q, k_ref[...], (((1,), (1,)), ((), ())),
                             preferred_element_type=F32)
             + lax.dot_general(qr, kr_ref[...], (((1,), (1,)), ((), ())),
                               preferred_element_type=F32)) * SCALE
        m = jnp.max(s, axis=-1, keepdims=True)
        p = jnp.exp(s - m)
        p = p / jnp.sum(p, axis=-1, keepdims=True)
        o_ref[...] = jnp.dot(p, v_ref[...], preferred_element_type=F32)

    return pl.pallas_call(
        body,
        grid=(B, H),
        out_shape=jax.ShapeDtypeStruct((B * S, D), F32),
        in_specs=[
            pl.BlockSpec((1, S, D), lambda b, h: (b, 0, 0)),
            pl.BlockSpec((D, DH), lambda b, h: (0, h)),
            pl.BlockSpec((1, D, DR), lambda b, h: (h, 0, 0)),
            pl.BlockSpec((S, DH), lambda b, h: (b, h)),
            pl.BlockSpec((S, DH), lambda b, h: (b, h)),
            pl.BlockSpec((S, DR), lambda b, h: (b, 0)),
        ],
        out_specs=pl.BlockSpec((S, DH), lambda b, h: (b, h)),
    )(x, Wq, Wqr_t, K, V, Kr)


def _out_proj(O, Wo):
    def body(o_ref, wo_ref, out_ref):
        out_ref[...] = jnp.dot(o_ref[...], wo_ref[...],
                               preferred_element_type=F32)

    RT, CT = 2, 4
    return pl.pallas_call(
        body,
        grid=(RT, CT),
        out_shape=jax.ShapeDtypeStruct((B * S, D), F32),
        in_specs=[
            pl.BlockSpec((B * S // RT, D), lambda i, j: (i, 0)),
            pl.BlockSpec((D, D // CT), lambda i, j: (0, j)),
        ],
        out_specs=pl.BlockSpec((B * S // RT, D // CT), lambda i, j: (i, j)),
    )(O, Wo)


def kernel(x, Wdkv, Wuk, Wuv, Wq, Wqr, Wkr, Wo):
    K, V, Kr = _kv_collective(x, Wdkv, Wuk, Wuv, Wkr)
    Wqr_t = jnp.transpose(Wqr.reshape(D, H, DR), (1, 0, 2))
    O = _attention(x, Wq, Wqr_t, K, V, Kr)
    out = _out_proj(O, Wo)
    return out.reshape(B, S, D)
